# Initial kernel scaffold; baseline (speedup 1.0000x reference)
import jax
import jax.numpy as jnp
from jax.experimental import pallas as pl


def kernel(x, emb):
    raise NotImplementedError("write your pallas kernel here")


# speedup vs baseline: 3.5666x; 3.5666x over previous
import jax
import jax.numpy as jnp
from jax.experimental import pallas as pl


def _copy_body(emb_ref, out_ref):
    out_ref[...] = emb_ref[...]


def kernel(x, emb):
    L = x.shape[1]
    D = emb.shape[1]
    rows_per_block = 256
    grid = (L // rows_per_block,)
    out = pl.pallas_call(
        _copy_body,
        out_shape=jax.ShapeDtypeStruct((L, D), emb.dtype),
        grid=grid,
        in_specs=[pl.BlockSpec((rows_per_block, D), lambda i: (i, 0))],
        out_specs=pl.BlockSpec((rows_per_block, D), lambda i: (i, 0)),
    )(emb)
    return out[None]
